# baseline (speedup 1.0000x reference)
import functools

import jax
import jax.numpy as jnp
from jax import lax
from jax.experimental import pallas as pl
from jax.experimental.pallas import tpu as pltpu
from jax.experimental.pallas import tpu_sc as plsc

B = 4096
F = 100
V = 10000
D = 100

NC = 2
NS = 16
NW = NC * NS
LANES = 16

ROWS = B * F
RPW = ROWS // NW
NG = RPW // LANES
PIPE = 8


def _body(batch_hbm, table_hbm, out_hbm, idx_v, sem):
    wid = lax.axis_index("s") * NC + lax.axis_index("c")
    base = wid * RPW
    lane = lax.iota(jnp.int32, LANES)

    pltpu.sync_copy(batch_hbm.at[pl.ds(base, RPW)], idx_v)

    def idx_step(i, _):
        off = i * LANES
        rid = base + off + lane
        f = lax.rem(rid, F)
        idx_v[pl.ds(off, LANES)] = idx_v[pl.ds(off, LANES)] + f * V
        return 0

    lax.fori_loop(0, NG, idx_step, 0)

    def gather_step(i, _):
        @pl.when(i < NG)
        def _fire():
            off = i * LANES
            iv = idx_v[pl.ds(off, LANES)]
            for j in range(LANES):
                pltpu.async_copy(
                    table_hbm.at[pl.ds(iv[j], 1), :],
                    out_hbm.at[pl.ds(base + off + j, 1), :],
                    sem,
                )

        @pl.when(i >= PIPE)
        def _drain():
            for j in range(LANES):
                pltpu.make_async_copy(
                    table_hbm.at[pl.ds(0, 1), :],
                    out_hbm.at[pl.ds(base, 1), :],
                    sem,
                ).wait()

        return 0

    lax.fori_loop(0, NG + PIPE, gather_step, 0)


@functools.partial(
    pl.kernel,
    mesh=plsc.VectorSubcoreMesh(core_axis_name="c", subcore_axis_name="s"),
    out_type=jax.ShapeDtypeStruct((ROWS, D), jnp.float32),
    scratch_types=[
        pltpu.VMEM((RPW,), jnp.int32),
        pltpu.SemaphoreType.DMA,
    ],
    compiler_params=pltpu.CompilerParams(use_tc_tiling_on_sc=True),
)
def _gather_kernel(batch_hbm, table_hbm, out_hbm, idx_v, sem):
    _body(batch_hbm, table_hbm, out_hbm, idx_v, sem)


def kernel(batch, tables):
    batch_flat = batch.reshape(ROWS)
    table_flat = tables.reshape(F * V, D)
    out = _gather_kernel(batch_flat, table_flat)
    return out.reshape(B, F, D)


# speedup vs baseline: 7.8297x; 7.8297x over previous
import functools

import jax
import jax.numpy as jnp
from jax import lax
from jax.experimental import pallas as pl
from jax.experimental.pallas import tpu as pltpu
from jax.experimental.pallas import tpu_sc as plsc

B = 4096
F = 100
V = 10000
D = 100
DP = 128

NC = 2
NS = 16
NW = NC * NS
LANES = 16

VC = V
BC = 1024
BPW = B // NW

_MESH = plsc.VectorSubcoreMesh(core_axis_name="c", subcore_axis_name="s")
_SC_PARAMS = pltpu.CompilerParams(use_tc_tiling_on_sc=True)


def _rowize_body(tab_ref, eye_ref, out_ref):
    slab = tab_ref[0]
    out_ref[...] = lax.dot_general(slab, eye_ref[...], (((0,), (0,)), ((), ())),
                                   precision=lax.Precision.HIGHEST,
                                   preferred_element_type=jnp.float32)


def _rowize(tab_nat, eye_dp):
    return pl.pallas_call(
        _rowize_body,
        grid=(F,),
        in_specs=[
            pl.BlockSpec((1, D, VC), lambda f: (f, 0, 0)),
            pl.BlockSpec((D, DP), lambda f: (0, 0)),
        ],
        out_specs=pl.BlockSpec((VC, DP), lambda f: (f, 0)),
        out_shape=jax.ShapeDtypeStruct((F * V, DP), jnp.float32),
    )(tab_nat, eye_dp)


def _gather_body(bat_nat, rows1, rows2, ibuf, idxb, rbuf, sem):
    wid = lax.axis_index("s") * NC + lax.axis_index("c")
    b0 = wid * BPW

    def f_step(f, _):
        pltpu.sync_copy(bat_nat.at[pl.ds(f, 1), pl.ds(b0, BPW)], ibuf)
        base = f * V

        def g_step(g, _):
            off = g * LANES
            idxb[pl.ds(off, LANES)] = ibuf[0, pl.ds(off, LANES)] + base
            return 0

        lax.fori_loop(0, BPW // LANES, g_step, 0)
        pltpu.async_copy(rows1.at[idxb], rbuf, sem).wait()
        pltpu.sync_copy(rbuf, rows2.at[pl.ds(f * B + b0, BPW), :])
        return 0

    lax.fori_loop(0, F, f_step, 0)


@functools.partial(
    pl.kernel,
    mesh=_MESH,
    out_type=jax.ShapeDtypeStruct((F * B, DP), jnp.float32),
    scratch_types=[
        pltpu.VMEM((1, BPW), jnp.int32),
        pltpu.VMEM((BPW,), jnp.int32),
        pltpu.VMEM((BPW, DP), jnp.float32),
        pltpu.SemaphoreType.DMA,
    ],
    compiler_params=_SC_PARAMS,
)
def _gather_kernel(bat_nat, rows1, rows2, ibuf, idxb, rbuf, sem):
    _gather_body(bat_nat, rows1, rows2, ibuf, idxb, rbuf, sem)


def _unslab_body(rows_ref, eye_ref, out_ref):
    chunk = rows_ref[...]
    slab = lax.dot_general(eye_ref[...], chunk, (((1,), (1,)), ((), ())),
                           precision=lax.Precision.HIGHEST,
                           preferred_element_type=jnp.float32)
    out_ref[...] = slab.reshape(1, D, BC)


def _unslab(rows2, eye_dp):
    return pl.pallas_call(
        _unslab_body,
        grid=(F, B // BC),
        in_specs=[
            pl.BlockSpec((BC, DP), lambda f, c: (f * (B // BC) + c, 0)),
            pl.BlockSpec((D, DP), lambda f, c: (0, 0)),
        ],
        out_specs=pl.BlockSpec((1, D, BC), lambda f, c: (f, 0, c)),
        out_shape=jax.ShapeDtypeStruct((F, D, B), jnp.float32),
    )(rows2, eye_dp)


def kernel(batch, tables):
    tab_nat = jnp.transpose(tables, (0, 2, 1))
    bat_nat = jnp.transpose(batch, (1, 0))
    eye_dp = jnp.eye(D, DP, dtype=jnp.float32)
    rows1 = _rowize(tab_nat, eye_dp)
    rows2 = _gather_kernel(bat_nat, rows1)
    out_nat = _unslab(rows2, eye_dp)
    return jnp.transpose(out_nat, (2, 0, 1))


# speedup vs baseline: 9.1976x; 1.1747x over previous
import functools

import jax
import jax.numpy as jnp
from jax import lax
from jax.experimental import pallas as pl
from jax.experimental.pallas import tpu as pltpu
from jax.experimental.pallas import tpu_sc as plsc

B = 4096
F = 100
V = 10000
D = 100
DP = 128

NC = 2
NS = 16
NW = NC * NS
LANES = 16

VC = V
BC = B
BPW = B // NW

_MESH = plsc.VectorSubcoreMesh(core_axis_name="c", subcore_axis_name="s")
_SC_PARAMS = pltpu.CompilerParams(use_tc_tiling_on_sc=True)


def _rowize_body(tab_ref, eye_ref, out_ref):
    slab = tab_ref[0]
    out_ref[...] = lax.dot_general(slab, eye_ref[...], (((0,), (0,)), ((), ())),
                                   precision=lax.Precision.HIGHEST,
                                   preferred_element_type=jnp.float32)


def _rowize(tab_nat, eye_dp):
    return pl.pallas_call(
        _rowize_body,
        grid=(F,),
        in_specs=[
            pl.BlockSpec((1, D, VC), lambda f: (f, 0, 0)),
            pl.BlockSpec((D, DP), lambda f: (0, 0)),
        ],
        out_specs=pl.BlockSpec((VC, DP), lambda f: (f, 0)),
        out_shape=jax.ShapeDtypeStruct((F * V, DP), jnp.float32),
    )(tab_nat, eye_dp)


def _gather_body(bat_nat, rows1, rows2, ibuf, idxb, rbuf, sem):
    wid = lax.axis_index("s") * NC + lax.axis_index("c")
    b0 = wid * BPW

    def f_step(f, _):
        pltpu.sync_copy(bat_nat.at[pl.ds(f, 1), pl.ds(b0, BPW)], ibuf)
        base = f * V

        def g_step(g, _):
            off = g * LANES
            idxb[pl.ds(off, LANES)] = ibuf[0, pl.ds(off, LANES)] + base
            return 0

        lax.fori_loop(0, BPW // LANES, g_step, 0)
        pltpu.async_copy(rows1.at[idxb], rbuf, sem).wait()
        pltpu.sync_copy(rbuf, rows2.at[pl.ds(f * B + b0, BPW), :])
        return 0

    lax.fori_loop(0, F, f_step, 0)


@functools.partial(
    pl.kernel,
    mesh=_MESH,
    out_type=jax.ShapeDtypeStruct((F * B, DP), jnp.float32),
    scratch_types=[
        pltpu.VMEM((1, BPW), jnp.int32),
        pltpu.VMEM((BPW,), jnp.int32),
        pltpu.VMEM((BPW, DP), jnp.float32),
        pltpu.SemaphoreType.DMA,
    ],
    compiler_params=_SC_PARAMS,
)
def _gather_kernel(bat_nat, rows1, rows2, ibuf, idxb, rbuf, sem):
    _gather_body(bat_nat, rows1, rows2, ibuf, idxb, rbuf, sem)


def _unslab_body(rows_ref, eye_ref, out_ref):
    chunk = rows_ref[...]
    slab = lax.dot_general(eye_ref[...], chunk, (((1,), (1,)), ((), ())),
                           precision=lax.Precision.HIGHEST,
                           preferred_element_type=jnp.float32)
    out_ref[...] = slab.reshape(1, D, BC)


def _unslab(rows2, eye_dp):
    return pl.pallas_call(
        _unslab_body,
        grid=(F,),
        in_specs=[
            pl.BlockSpec((BC, DP), lambda f: (f, 0)),
            pl.BlockSpec((D, DP), lambda f: (0, 0)),
        ],
        out_specs=pl.BlockSpec((1, D, BC), lambda f: (f, 0, 0)),
        out_shape=jax.ShapeDtypeStruct((F, D, B), jnp.float32),
    )(rows2, eye_dp)


def kernel(batch, tables):
    tab_nat = jnp.transpose(tables, (0, 2, 1))
    bat_nat = jnp.transpose(batch, (1, 0))
    eye_dp = jnp.eye(D, DP, dtype=jnp.float32)
    rows1 = _rowize(tab_nat, eye_dp)
    rows2 = _gather_kernel(bat_nat, rows1)
    out_nat = _unslab(rows2, eye_dp)
    return jnp.transpose(out_nat, (2, 0, 1))


# speedup vs baseline: 10.1587x; 1.1045x over previous
import functools

import jax
import jax.numpy as jnp
from jax import lax
from jax.experimental import pallas as pl
from jax.experimental.pallas import tpu as pltpu
from jax.experimental.pallas import tpu_sc as plsc

B = 4096
F = 100
V = 10000
D = 100
DP = 128

NC = 2
NS = 16
NW = NC * NS
LANES = 16

VC = V
FB = 1
BC = B
BPW = B // NW

_MESH = plsc.VectorSubcoreMesh(core_axis_name="c", subcore_axis_name="s")
_SC_PARAMS = pltpu.CompilerParams(use_tc_tiling_on_sc=True)


def _rowize_body(tab_ref, eye_ref, out_ref):
    eye = eye_ref[...]
    for k in range(FB):
        slab = tab_ref[k]
        out_ref[pl.ds(k * VC, VC), :] = lax.dot_general(
            slab, eye, (((0,), (0,)), ((), ())),
            precision=lax.Precision.HIGHEST,
            preferred_element_type=jnp.float32)


def _rowize(tab_nat, eye_dp):
    return pl.pallas_call(
        _rowize_body,
        grid=(F // FB,),
        in_specs=[
            pl.BlockSpec((FB, D, VC), lambda f: (f, 0, 0)),
            pl.BlockSpec((D, DP), lambda f: (0, 0)),
        ],
        out_specs=pl.BlockSpec((FB * VC, DP), lambda f: (f, 0)),
        out_shape=jax.ShapeDtypeStruct((F * V, DP), jnp.float32),
    )(tab_nat, eye_dp)


def _gather_body(bat_nat, rows1, rows2, ibufs, idxbs, rbufs, gsems, wsems):
    wid = lax.axis_index("s") * NC + lax.axis_index("c")
    b0 = wid * BPW

    def prep(f, ibuf, idxb, rbuf, gsem):
        pltpu.sync_copy(bat_nat.at[pl.ds(f, 1), pl.ds(b0, BPW)], ibuf)
        base = f * V

        def g_step(g, _):
            off = g * LANES
            idxb[pl.ds(off, LANES)] = ibuf[0, pl.ds(off, LANES)] + base
            return 0

        lax.fori_loop(0, BPW // LANES, g_step, 0)
        return pltpu.async_copy(rows1.at[idxb], rbuf, gsem)

    def pair_step(p, _):
        f0 = 2 * p
        cps = [prep(f0 + k, ibufs[k], idxbs[k], rbufs[k], gsems[k])
               for k in range(2)]
        wps = []
        for k in range(2):
            cps[k].wait()
            wps.append(pltpu.async_copy(
                rbufs[k], rows2.at[pl.ds((f0 + k) * B + b0, BPW), :],
                wsems[k]))
        for k in range(2):
            wps[k].wait()
        return 0

    lax.fori_loop(0, F // 2, pair_step, 0)


@functools.partial(
    pl.kernel,
    mesh=_MESH,
    out_type=jax.ShapeDtypeStruct((F * B, DP), jnp.float32),
    scratch_types=[
        pltpu.VMEM((1, BPW), jnp.int32),
        pltpu.VMEM((1, BPW), jnp.int32),
        pltpu.VMEM((BPW,), jnp.int32),
        pltpu.VMEM((BPW,), jnp.int32),
        pltpu.VMEM((BPW, DP), jnp.float32),
        pltpu.VMEM((BPW, DP), jnp.float32),
        pltpu.SemaphoreType.DMA,
        pltpu.SemaphoreType.DMA,
        pltpu.SemaphoreType.DMA,
        pltpu.SemaphoreType.DMA,
    ],
    compiler_params=_SC_PARAMS,
)
def _gather_kernel(bat_nat, rows1, rows2, ib0, ib1, ix0, ix1, rb0, rb1,
                   gs0, gs1, ws0, ws1):
    _gather_body(bat_nat, rows1, rows2, (ib0, ib1), (ix0, ix1), (rb0, rb1),
                 (gs0, gs1), (ws0, ws1))


def _unslab_body(rows_ref, eye_ref, out_ref):
    chunk = rows_ref[...]
    slab = lax.dot_general(eye_ref[...], chunk, (((1,), (1,)), ((), ())),
                           precision=lax.Precision.HIGHEST,
                           preferred_element_type=jnp.float32)
    out_ref[...] = slab.reshape(1, D, BC)


def _unslab(rows2, eye_dp):
    return pl.pallas_call(
        _unslab_body,
        grid=(F,),
        in_specs=[
            pl.BlockSpec((BC, DP), lambda f: (f, 0)),
            pl.BlockSpec((D, DP), lambda f: (0, 0)),
        ],
        out_specs=pl.BlockSpec((1, D, BC), lambda f: (f, 0, 0)),
        out_shape=jax.ShapeDtypeStruct((F, D, B), jnp.float32),
    )(rows2, eye_dp)


def kernel(batch, tables):
    tab_nat = jnp.transpose(tables, (0, 2, 1))
    bat_nat = jnp.transpose(batch, (1, 0))
    eye_dp = jnp.eye(D, DP, dtype=jnp.float32)
    rows1 = _rowize(tab_nat, eye_dp)
    rows2 = _gather_kernel(bat_nat, rows1)
    out_nat = _unslab(rows2, eye_dp)
    return jnp.transpose(out_nat, (2, 0, 1))


# speedup vs baseline: 10.5412x; 1.0377x over previous
import functools

import jax
import jax.numpy as jnp
from jax import lax
from jax.experimental import pallas as pl
from jax.experimental.pallas import tpu as pltpu
from jax.experimental.pallas import tpu_sc as plsc

B = 4096
F = 100
V = 10000
D = 100
DP = 128

NC = 2
NS = 16
NW = NC * NS
LANES = 16

VC = V
FB = 1
BC = B
BPW = B // NW

_MESH = plsc.VectorSubcoreMesh(core_axis_name="c", subcore_axis_name="s")
_SC_PARAMS = pltpu.CompilerParams(use_tc_tiling_on_sc=True)


def _rowize_body(tab_ref, eye_ref, out_ref):
    eye = eye_ref[...]
    for k in range(FB):
        slab = tab_ref[k]
        out_ref[pl.ds(k * VC, VC), :] = lax.dot_general(
            slab, eye, (((0,), (0,)), ((), ())),
            precision=lax.Precision.HIGHEST,
            preferred_element_type=jnp.float32)


def _rowize(tab_nat, eye_dp):
    return pl.pallas_call(
        _rowize_body,
        grid=(F // FB,),
        in_specs=[
            pl.BlockSpec((FB, D, VC), lambda f: (f, 0, 0)),
            pl.BlockSpec((D, DP), lambda f: (0, 0)),
        ],
        out_specs=pl.BlockSpec((FB * VC, DP), lambda f: (f, 0)),
        out_shape=jax.ShapeDtypeStruct((F * V, DP), jnp.float32),
    )(tab_nat, eye_dp)


def _gather_body(bat_nat, rows1, rows2, iball, idxbs, rbufs, gsems, wsems):
    wid = lax.axis_index("s") * NC + lax.axis_index("c")
    b0 = wid * BPW
    pltpu.sync_copy(bat_nat.at[:, pl.ds(b0, BPW)], iball)

    def prep(f, idxb, rbuf, gsem):
        base = f * V

        def g_step(g, _):
            off = g * LANES
            idxb[pl.ds(off, LANES)] = iball[f, pl.ds(off, LANES)] + base
            return 0

        lax.fori_loop(0, BPW // LANES, g_step, 0)
        return pltpu.async_copy(rows1.at[idxb], rbuf, gsem)

    NB = 4

    def quad_step(p, _):
        f0 = NB * p
        cps = [prep(f0 + k, idxbs[k], rbufs[k], gsems[k])
               for k in range(NB)]
        wps = []
        for k in range(NB):
            cps[k].wait()
            wps.append(pltpu.async_copy(
                rbufs[k], rows2.at[pl.ds((f0 + k) * B + b0, BPW), :],
                wsems[k]))
        for k in range(NB):
            wps[k].wait()
        return 0

    lax.fori_loop(0, F // NB, quad_step, 0)


@functools.partial(
    pl.kernel,
    mesh=_MESH,
    out_type=jax.ShapeDtypeStruct((F * B, DP), jnp.float32),
    scratch_types=[
        pltpu.VMEM((F, BPW), jnp.int32),
        [pltpu.VMEM((BPW,), jnp.int32)] * 4,
        [pltpu.VMEM((BPW, DP), jnp.float32)] * 4,
        [pltpu.SemaphoreType.DMA] * 4,
        [pltpu.SemaphoreType.DMA] * 4,
    ],
    compiler_params=_SC_PARAMS,
)
def _gather_kernel(bat_nat, rows1, rows2, iball, idxbs, rbufs, gsems, wsems):
    _gather_body(bat_nat, rows1, rows2, iball, idxbs, rbufs, gsems, wsems)


def _unslab_body(rows_ref, eye_ref, out_ref):
    chunk = rows_ref[...]
    slab = lax.dot_general(eye_ref[...], chunk, (((1,), (1,)), ((), ())),
                           precision=lax.Precision.HIGHEST,
                           preferred_element_type=jnp.float32)
    out_ref[...] = slab.reshape(1, D, BC)


def _unslab(rows2, eye_dp):
    return pl.pallas_call(
        _unslab_body,
        grid=(F,),
        in_specs=[
            pl.BlockSpec((BC, DP), lambda f: (f, 0)),
            pl.BlockSpec((D, DP), lambda f: (0, 0)),
        ],
        out_specs=pl.BlockSpec((1, D, BC), lambda f: (f, 0, 0)),
        out_shape=jax.ShapeDtypeStruct((F, D, B), jnp.float32),
    )(rows2, eye_dp)


def kernel(batch, tables):
    tab_nat = jnp.transpose(tables, (0, 2, 1))
    bat_nat = jnp.transpose(batch, (1, 0))
    eye_dp = jnp.eye(D, DP, dtype=jnp.float32)
    rows1 = _rowize(tab_nat, eye_dp)
    rows2 = _gather_kernel(bat_nat, rows1)
    out_nat = _unslab(rows2, eye_dp)
    return jnp.transpose(out_nat, (2, 0, 1))


# speedup vs baseline: 10.7863x; 1.0232x over previous
import functools

import jax
import jax.numpy as jnp
from jax import lax
from jax.experimental import pallas as pl
from jax.experimental.pallas import tpu as pltpu
from jax.experimental.pallas import tpu_sc as plsc

B = 4096
F = 100
V = 10000
D = 100
DP = 128

NC = 2
NS = 16
NW = NC * NS
LANES = 16

VC = V
FB = 1
BC = B
FB3 = 2
BPW = B // NW

_MESH = plsc.VectorSubcoreMesh(core_axis_name="c", subcore_axis_name="s")
_SC_PARAMS = pltpu.CompilerParams(use_tc_tiling_on_sc=True)


def _rowize_body(tab_ref, eye_ref, out_ref):
    eye = eye_ref[...]
    for k in range(FB):
        slab = tab_ref[k]
        out_ref[pl.ds(k * VC, VC), :] = lax.dot_general(
            slab, eye, (((0,), (0,)), ((), ())),
            precision=lax.Precision.HIGHEST,
            preferred_element_type=jnp.float32)


def _rowize(tab_nat, eye_dp):
    return pl.pallas_call(
        _rowize_body,
        grid=(F // FB,),
        in_specs=[
            pl.BlockSpec((FB, D, VC), lambda f: (f, 0, 0)),
            pl.BlockSpec((D, DP), lambda f: (0, 0)),
        ],
        out_specs=pl.BlockSpec((FB * VC, DP), lambda f: (f, 0)),
        out_shape=jax.ShapeDtypeStruct((F * V, DP), jnp.float32),
    )(tab_nat, eye_dp)


def _gather_body(bat_nat, rows1, rows2, iball, idxbs, rbufs, gsems, wsems):
    wid = lax.axis_index("s") * NC + lax.axis_index("c")
    b0 = wid * BPW
    pltpu.sync_copy(bat_nat.at[:, pl.ds(b0, BPW)], iball)

    def prep(f, idxb, rbuf, gsem):
        base = f * V

        def g_step(g, _):
            off = g * LANES
            idxb[pl.ds(off, LANES)] = iball[f, pl.ds(off, LANES)] + base
            return 0

        lax.fori_loop(0, BPW // LANES, g_step, 0)
        return pltpu.async_copy(rows1.at[idxb], rbuf, gsem)

    NB = 4

    def quad_step(p, _):
        f0 = NB * p
        cps = [prep(f0 + k, idxbs[k], rbufs[k], gsems[k])
               for k in range(NB)]
        wps = []
        for k in range(NB):
            cps[k].wait()
            wps.append(pltpu.async_copy(
                rbufs[k], rows2.at[pl.ds((f0 + k) * B + b0, BPW), :],
                wsems[k]))
        for k in range(NB):
            wps[k].wait()
        return 0

    lax.fori_loop(0, F // NB, quad_step, 0)


@functools.partial(
    pl.kernel,
    mesh=_MESH,
    out_type=jax.ShapeDtypeStruct((F * B, DP), jnp.float32),
    scratch_types=[
        pltpu.VMEM((F, BPW), jnp.int32),
        [pltpu.VMEM((BPW,), jnp.int32)] * 4,
        [pltpu.VMEM((BPW, DP), jnp.float32)] * 4,
        [pltpu.SemaphoreType.DMA] * 4,
        [pltpu.SemaphoreType.DMA] * 4,
    ],
    compiler_params=_SC_PARAMS,
)
def _gather_kernel(bat_nat, rows1, rows2, iball, idxbs, rbufs, gsems, wsems):
    _gather_body(bat_nat, rows1, rows2, iball, idxbs, rbufs, gsems, wsems)


def _unslab_body(rows_ref, eye_ref, out_ref):
    eye = eye_ref[...]
    for k in range(FB3):
        chunk = rows_ref[pl.ds(k * BC, BC), :]
        slab = lax.dot_general(eye, chunk, (((1,), (1,)), ((), ())),
                               precision=lax.Precision.HIGHEST,
                               preferred_element_type=jnp.float32)
        out_ref[k] = slab


def _unslab(rows2, eye_dp):
    return pl.pallas_call(
        _unslab_body,
        grid=(F // FB3,),
        in_specs=[
            pl.BlockSpec((FB3 * BC, DP), lambda f: (f, 0)),
            pl.BlockSpec((D, DP), lambda f: (0, 0)),
        ],
        out_specs=pl.BlockSpec((FB3, D, BC), lambda f: (f, 0, 0)),
        out_shape=jax.ShapeDtypeStruct((F, D, B), jnp.float32),
    )(rows2, eye_dp)


def kernel(batch, tables):
    tab_nat = jnp.transpose(tables, (0, 2, 1))
    bat_nat = jnp.transpose(batch, (1, 0))
    eye_dp = jnp.eye(D, DP, dtype=jnp.float32)
    rows1 = _rowize(tab_nat, eye_dp)
    rows2 = _gather_kernel(bat_nat, rows1)
    out_nat = _unslab(rows2, eye_dp)
    return jnp.transpose(out_nat, (2, 0, 1))
